# baseline (speedup 1.0000x reference)
import functools

import jax
import jax.numpy as jnp
import numpy as np
from jax import lax
from jax.experimental import pallas as pl
from jax.experimental.pallas import tpu as pltpu
from jax.experimental.pallas import tpu_sc as plsc

_N = 10000
_D = 128
_E = 320000
_OUT_DIM = 103
_MIN_NORM = 1e-15
_MAXNORM = 1.0 - 1e-5

_NCORE = 2
_NSUB = 16
_HD = _D // _NCORE
_CHUNK = 128
_CPT = 160
_G = 80
_EPW = _CHUNK * _CPT
_EPAD = _NSUB * _EPW
_NPAD = 10240
_RPT = _NPAD // _NSUB
_ZROWS = 128
_NGBUF = 4
_NSBUF = 2
_STG = 80

_QHALF = np.concatenate(
    [np.concatenate([g * 32 + 2 * np.arange(16),
                     g * 32 + 2 * np.arange(16) + 1]) for g in range(2)])
_PFULL = np.concatenate([_QHALF, 64 + _QHALF])

_BLK = 1000
_NBLK = _N // _BLK


def _norm(x):
    return jnp.maximum(jnp.sqrt(jnp.sum(x * x, axis=-1, keepdims=True)),
                       _MIN_NORM)


def _proj(x):
    n = _norm(x)
    return jnp.where(n > _MAXNORM, x / n * _MAXNORM, x)


def _artanh(x):
    x = jnp.clip(x, -1.0 + 1e-7, 1.0 - 1e-7)
    return 0.5 * jnp.log((1.0 + x) / (1.0 - x))


def _expmap0(u):
    n = _norm(u)
    return jnp.tanh(n) * u / n


def _logmap0(p):
    n = _norm(p)
    return _artanh(n) * p / n


def _mobius_add(x, y):
    x2 = jnp.sum(x * x, axis=-1, keepdims=True)
    y2 = jnp.sum(y * y, axis=-1, keepdims=True)
    xy = jnp.sum(x * y, axis=-1, keepdims=True)
    num = (1.0 + 2.0 * xy + y2) * x + (1.0 - x2) * y
    den = 1.0 + 2.0 * xy + x2 * y2
    return num / jnp.maximum(den, _MIN_NORM)


def _hyplinear_to_tangent(x, wt, b, proj_first):
    if proj_first:
        x = _proj(x)
    xn = _norm(x)
    mx = jnp.dot(x, wt, preferred_element_type=jnp.float32)
    mxn = _norm(mx)
    res = jnp.tanh(mxn / xn * _artanh(xn)) * mx / mxn
    mv = _proj(res)
    hyp_b = _proj(_expmap0(b))
    h = _proj(_mobius_add(mv, hyp_b))
    return _logmap0(h)


def _agg_to_hyp(p0, p1):
    agg = jnp.concatenate([p0, p1], axis=-1)
    h = _proj(_expmap0(agg))
    xt = jnp.maximum(_logmap0(h), 0.0)
    return _proj(_expmap0(xt))


def _stage_a_body(x_ref, wt_ref, b_ref, o_ref):
    res = _hyplinear_to_tangent(x_ref[...], wt_ref[...], b_ref[...],
                                proj_first=True)
    o_ref[...] = res.astype(jnp.bfloat16)


def _stage_c_body(p0_ref, p1_ref, wt_ref, b_ref, o_ref):
    h = _agg_to_hyp(p0_ref[...], p1_ref[...])
    res = _hyplinear_to_tangent(h, wt_ref[...], b_ref[...],
                                proj_first=False)
    o_ref[...] = res.astype(jnp.bfloat16)


def _stage_d_body(p0_ref, p1_ref, wt_ref, b_ref, o_ref):
    h = _agg_to_hyp(p0_ref[...], p1_ref[...])
    z = _logmap0(h)
    o_ref[...] = jnp.dot(z, wt_ref[...],
                         preferred_element_type=jnp.float32) + b_ref[...]


_row_spec = pl.BlockSpec((_BLK, _D), lambda i: (i, 0))
_half_spec = pl.BlockSpec((_BLK, _HD), lambda i: (i, 0))
_w_spec = pl.BlockSpec((_D, _D), lambda i: (0, 0))
_b_spec = pl.BlockSpec((1, _D), lambda i: (0, 0))
_out_struct = jax.ShapeDtypeStruct((_N, _D), jnp.float32)
_bf_struct = jax.ShapeDtypeStruct((_N, _D), jnp.bfloat16)

_stage_a = pl.pallas_call(
    _stage_a_body, grid=(_NBLK,),
    in_specs=[_row_spec, _w_spec, _b_spec],
    out_specs=_row_spec, out_shape=_bf_struct)

_stage_c = pl.pallas_call(
    _stage_c_body, grid=(_NBLK,),
    in_specs=[_half_spec, _half_spec, _w_spec, _b_spec],
    out_specs=_row_spec, out_shape=_bf_struct)

_stage_d = pl.pallas_call(
    _stage_d_body, grid=(_NBLK,),
    in_specs=[_half_spec, _half_spec, _w_spec, _b_spec],
    out_specs=_row_spec, out_shape=_out_struct)


def _spmm_sc(xt2, srcp, dstp, ewp):
    mesh = plsc.VectorSubcoreMesh(core_axis_name="c", subcore_axis_name="s")

    @functools.partial(
        pl.kernel,
        compiler_params=pltpu.CompilerParams(use_tc_tiling_on_sc=False,
                                             needs_layout_passes=False),
        out_type=jax.ShapeDtypeStruct((_NCORE, _NPAD, _HD), jnp.float32),
        mesh=mesh,
        scratch_types=[
            pltpu.VMEM((_G, _CHUNK), jnp.int32),
            pltpu.VMEM((_G, _CHUNK), jnp.int32),
            pltpu.VMEM((_G, _CHUNK), jnp.float32),
            [pltpu.VMEM((_CHUNK, _HD), jnp.bfloat16)] * _NGBUF,
            [pltpu.VMEM((_CHUNK, _HD), jnp.float32)] * _NSBUF,
            pltpu.VMEM_SHARED((_NPAD, _HD), jnp.float32),
            [pltpu.SemaphoreType.DMA] * _NGBUF,
            [pltpu.SemaphoreType.DMA] * _NSBUF,
        ],
    )
    def k(xt_hbm, srcp_hbm, dstp_hbm, ewp_hbm, out_hbm,
          src_v, dst_v, ew_v, gbufs, sbufs, acc_sh, gsem, ssem):
        c = lax.axis_index("c")
        s = lax.axis_index("s")
        base = s * _RPT

        def zfill(i, carry):
            for q in range(_HD // 16):
                sbufs[0][i, pl.ds(q * 16, 16)] = jnp.zeros((16,),
                                                           jnp.float32)
            return carry
        lax.fori_loop(0, _ZROWS, zfill, 0)
        for r in range(_RPT // _ZROWS):
            pltpu.sync_copy(sbufs[0], acc_sh.at[pl.ds(base + r * _ZROWS,
                                                      _ZROWS)])
        plsc.subcore_barrier()

        mask_hi = jnp.full((16,), -65536, jnp.int32)

        def group_body(g, carry0):
            goff = g * _G
            pltpu.sync_copy(srcp_hbm.at[s, pl.ds(goff, _G)], src_v)
            pltpu.sync_copy(dstp_hbm.at[s, pl.ds(goff, _G)], dst_v)
            pltpu.sync_copy(ewp_hbm.at[s, pl.ds(goff, _G)], ew_v)

            def tbody(r, carry):
                for q in range(_CHUNK // 16):
                    sl = pl.ds(q * 16, 16)
                    v = src_v[r, sl]
                    src_v[r, sl] = v + v + c
                return carry
            lax.fori_loop(0, _G, tbody, 0)

            for pj in range(_NGBUF - 1):
                pltpu.async_copy(xt_hbm.at[src_v.at[pj]], gbufs[pj],
                                 gsem[pj])

            def quad(t, carry):
                for gb in range(_NGBUF):
                    j = t * _NGBUF + gb
                    sb = gb % _NSBUF
                    gn = (gb + _NGBUF - 1) % _NGBUF

                    @pl.when(j + _NGBUF - 1 < _G)
                    def _():
                        pltpu.async_copy(
                            xt_hbm.at[src_v.at[j + _NGBUF - 1]],
                            gbufs[gn], gsem[gn])

                    @pl.when(j >= _NSBUF)
                    def _():
                        pltpu.make_async_copy(
                            sbufs[sb], acc_sh.at[dst_v.at[j - _NSBUF]],
                            ssem[sb]).wait()

                    pltpu.make_async_copy(xt_hbm.at[src_v.at[j]],
                                          gbufs[gb], gsem[gb]).wait()

                    def group16(gg, c2):
                        wv = ew_v[j, pl.ds(gg * 16, 16)]
                        for l in range(16):
                            e = gg * 16 + l
                            w = wv[l]
                            for g2 in range(2):
                                xb = gbufs[gb][e, pl.ds(g2 * 32, 32)]
                                xi = plsc.bitcast(xb, jnp.int32)
                                lo = plsc.bitcast(xi << 16, jnp.float32)
                                hi = plsc.bitcast(xi & mask_hi, jnp.float32)
                                sbufs[sb][e, pl.ds(g2 * 32, 16)] = lo * w
                                sbufs[sb][e, pl.ds(g2 * 32 + 16, 16)] = (
                                    hi * w)
                        return c2
                    lax.fori_loop(0, _CHUNK // 16, group16, 0)

                    pltpu.async_copy(sbufs[sb], acc_sh.at[dst_v.at[j]],
                                     ssem[sb], add=True)
                return carry
            lax.fori_loop(0, _G // _NGBUF, quad, 0)

            pltpu.make_async_copy(sbufs[0], acc_sh.at[dst_v.at[_G - 2]],
                                  ssem[0]).wait()
            pltpu.make_async_copy(sbufs[1], acc_sh.at[dst_v.at[_G - 1]],
                                  ssem[1]).wait()
            return carry0
        lax.fori_loop(0, _CPT // _G, group_body, 0)

        plsc.subcore_barrier()
        pltpu.sync_copy(acc_sh.at[pl.ds(base, _RPT)],
                        out_hbm.at[c, pl.ds(base, _RPT)])

    return k(xt2, srcp, dstp, ewp)


def kernel(x, edge_index, edge_weight, W1, b1, W2, b2, W_out, b_out):
    pad = _EPAD - _E
    srcp = jnp.concatenate(
        [edge_index[0], jnp.zeros((pad,), jnp.int32)]).reshape(
            _NSUB, _CPT, _CHUNK)
    dstp = jnp.concatenate(
        [edge_index[1], jnp.zeros((pad,), jnp.int32)]).reshape(
            _NSUB, _CPT, _CHUNK)
    ewp = jnp.concatenate(
        [edge_weight, jnp.zeros((pad,), jnp.float32)]).reshape(
            _NSUB, _CPT, _CHUNK)

    w1t = W1.T
    w2t_p = W2.T[_PFULL, :]
    woutt_p = jnp.pad(W_out.T, ((0, 0), (0, _D - _OUT_DIM)))[_PFULL, :]
    boutp = jnp.pad(b_out, (0, _D - _OUT_DIM))

    xt = _stage_a(x, w1t, b1.reshape(1, _D))
    p = _spmm_sc(xt.reshape(2 * _N, _HD), srcp, dstp, ewp)
    xt = _stage_c(p[0, :_N], p[1, :_N], w2t_p, b2.reshape(1, _D))
    p = _spmm_sc(xt.reshape(2 * _N, _HD), srcp, dstp, ewp)
    out = _stage_d(p[0, :_N], p[1, :_N], woutt_p, boutp.reshape(1, _D))
    return out[:, :_OUT_DIM]


# speedup vs baseline: 1.1525x; 1.1525x over previous
import functools

import jax
import jax.numpy as jnp
import numpy as np
from jax import lax
from jax.experimental import pallas as pl
from jax.experimental.pallas import tpu as pltpu
from jax.experimental.pallas import tpu_sc as plsc

_N = 10000
_D = 128
_E = 320000
_OUT_DIM = 103
_MIN_NORM = 1e-15
_MAXNORM = 1.0 - 1e-5

_NCORE = 2
_NSUB = 16
_HD = _D // _NCORE
_CHUNK = 128
_CPT = 160
_G = 40
_EPW = _CHUNK * _CPT
_EPAD = _NSUB * _EPW
_NPAD = 10240
_RPT = _NPAD // _NSUB
_ZROWS = 128
_NGBUF = 4
_NSBUF = 2
_STG = 80

_QHALF = np.concatenate(
    [np.concatenate([g * 32 + 2 * np.arange(16),
                     g * 32 + 2 * np.arange(16) + 1]) for g in range(2)])
_PFULL = np.concatenate([_QHALF, 64 + _QHALF])

_BLK = 1000
_NBLK = _N // _BLK


def _norm(x):
    return jnp.maximum(jnp.sqrt(jnp.sum(x * x, axis=-1, keepdims=True)),
                       _MIN_NORM)


def _proj(x):
    n = _norm(x)
    return jnp.where(n > _MAXNORM, x / n * _MAXNORM, x)


def _artanh(x):
    x = jnp.clip(x, -1.0 + 1e-7, 1.0 - 1e-7)
    return 0.5 * jnp.log((1.0 + x) / (1.0 - x))


def _expmap0(u):
    n = _norm(u)
    return jnp.tanh(n) * u / n


def _logmap0(p):
    n = _norm(p)
    return _artanh(n) * p / n


def _mobius_add(x, y):
    x2 = jnp.sum(x * x, axis=-1, keepdims=True)
    y2 = jnp.sum(y * y, axis=-1, keepdims=True)
    xy = jnp.sum(x * y, axis=-1, keepdims=True)
    num = (1.0 + 2.0 * xy + y2) * x + (1.0 - x2) * y
    den = 1.0 + 2.0 * xy + x2 * y2
    return num / jnp.maximum(den, _MIN_NORM)


def _hyplinear_to_tangent(x, wt, b, proj_first):
    if proj_first:
        x = _proj(x)
    xn = _norm(x)
    mx = jnp.dot(x, wt, preferred_element_type=jnp.float32)
    mxn = _norm(mx)
    res = jnp.tanh(mxn / xn * _artanh(xn)) * mx / mxn
    mv = _proj(res)
    hyp_b = _proj(_expmap0(b))
    h = _proj(_mobius_add(mv, hyp_b))
    return _logmap0(h)


def _agg_to_hyp(p0, p1):
    agg = jnp.concatenate([p0, p1], axis=-1)
    h = _proj(_expmap0(agg))
    xt = jnp.maximum(_logmap0(h), 0.0)
    return _proj(_expmap0(xt))


def _stage_a_body(x_ref, wt_ref, b_ref, o0_ref, o1_ref):
    res = _hyplinear_to_tangent(x_ref[...], wt_ref[...], b_ref[...],
                                proj_first=True)
    o0_ref[...] = res[:, :_HD].astype(jnp.bfloat16)
    o1_ref[...] = res[:, _HD:].astype(jnp.bfloat16)


def _stage_c_body(p0_ref, p1_ref, wt_ref, b_ref, o0_ref, o1_ref):
    h = _agg_to_hyp(p0_ref[...], p1_ref[...])
    res = _hyplinear_to_tangent(h, wt_ref[...], b_ref[...],
                                proj_first=False)
    o0_ref[...] = res[:, :_HD].astype(jnp.bfloat16)
    o1_ref[...] = res[:, _HD:].astype(jnp.bfloat16)


def _stage_d_body(p0_ref, p1_ref, wt_ref, b_ref, o_ref):
    h = _agg_to_hyp(p0_ref[...], p1_ref[...])
    z = _logmap0(h)
    o_ref[...] = jnp.dot(z, wt_ref[...],
                         preferred_element_type=jnp.float32) + b_ref[...]


_row_spec = pl.BlockSpec((_BLK, _D), lambda i: (i, 0))
_half_spec = pl.BlockSpec((_BLK, _HD), lambda i: (i, 0))
_w_spec = pl.BlockSpec((_D, _D), lambda i: (0, 0))
_b_spec = pl.BlockSpec((1, _D), lambda i: (0, 0))
_out_struct = jax.ShapeDtypeStruct((_N, _D), jnp.float32)
_bf_struct = jax.ShapeDtypeStruct((_N, _HD), jnp.bfloat16)

_stage_a = pl.pallas_call(
    _stage_a_body, grid=(_NBLK,),
    in_specs=[_row_spec, _w_spec, _b_spec],
    out_specs=[_half_spec, _half_spec],
    out_shape=[_bf_struct, _bf_struct])

_stage_c = pl.pallas_call(
    _stage_c_body, grid=(_NBLK,),
    in_specs=[_half_spec, _half_spec, _w_spec, _b_spec],
    out_specs=[_half_spec, _half_spec],
    out_shape=[_bf_struct, _bf_struct])

_stage_d = pl.pallas_call(
    _stage_d_body, grid=(_NBLK,),
    in_specs=[_half_spec, _half_spec, _w_spec, _b_spec],
    out_specs=_row_spec, out_shape=_out_struct)


def _spmm_sc(xta, xtb, srcp, dstp, ewp):
    mesh = plsc.VectorSubcoreMesh(core_axis_name="c", subcore_axis_name="s")

    @functools.partial(
        pl.kernel,
        compiler_params=pltpu.CompilerParams(use_tc_tiling_on_sc=False,
                                             needs_layout_passes=False),
        out_type=jax.ShapeDtypeStruct((_NCORE, _NPAD, _HD), jnp.float32),
        mesh=mesh,
        scratch_types=[
            pltpu.VMEM((_G, _CHUNK), jnp.int32),
            pltpu.VMEM((_G, _CHUNK), jnp.int32),
            pltpu.VMEM((_G, _CHUNK), jnp.float32),
            [pltpu.VMEM((_CHUNK, _HD), jnp.bfloat16)] * _NGBUF,
            [pltpu.VMEM((_CHUNK, _HD), jnp.float32)] * _NSBUF,
            pltpu.VMEM_SHARED((_N, _HD), jnp.bfloat16),
            pltpu.VMEM_SHARED((_NPAD, _HD), jnp.float32),
            [pltpu.SemaphoreType.DMA] * _NGBUF,
            [pltpu.SemaphoreType.DMA] * _NSBUF,
        ],
    )
    def k(xta_hbm, xtb_hbm, srcp_hbm, dstp_hbm, ewp_hbm, out_hbm,
          src_v, dst_v, ew_v, gbufs, sbufs, xt_sh, acc_sh, gsem, ssem):
        c = lax.axis_index("c")
        s = lax.axis_index("s")
        base = s * _RPT

        for it in range(_RPT // _STG):
            row0 = base + it * _STG

            @pl.when(row0 < _N)
            def _():
                @pl.when(c == 0)
                def _():
                    pltpu.sync_copy(xta_hbm.at[pl.ds(row0, _STG)],
                                    xt_sh.at[pl.ds(row0, _STG)])

                @pl.when(c == 1)
                def _():
                    pltpu.sync_copy(xtb_hbm.at[pl.ds(row0, _STG)],
                                    xt_sh.at[pl.ds(row0, _STG)])

        def zfill(i, carry):
            for q in range(_HD // 16):
                sbufs[0][i, pl.ds(q * 16, 16)] = jnp.zeros((16,),
                                                           jnp.float32)
            return carry
        lax.fori_loop(0, _ZROWS, zfill, 0)
        for r in range(_RPT // _ZROWS):
            pltpu.sync_copy(sbufs[0], acc_sh.at[pl.ds(base + r * _ZROWS,
                                                      _ZROWS)])
        plsc.subcore_barrier()

        mask_hi = jnp.full((16,), -65536, jnp.int32)

        def group_body(g, carry0):
            goff = g * _G
            pltpu.sync_copy(srcp_hbm.at[s, pl.ds(goff, _G)], src_v)
            pltpu.sync_copy(dstp_hbm.at[s, pl.ds(goff, _G)], dst_v)
            pltpu.sync_copy(ewp_hbm.at[s, pl.ds(goff, _G)], ew_v)

            for pj in range(_NGBUF - 1):
                if pj % 2 == 0:
                    pltpu.async_copy(xt_sh.at[src_v.at[pj]], gbufs[pj],
                                     gsem[pj])
                else:
                    @pl.when(c == 0)
                    def _():
                        pltpu.async_copy(xta_hbm.at[src_v.at[pj]],
                                         gbufs[pj], gsem[pj])

                    @pl.when(c == 1)
                    def _():
                        pltpu.async_copy(xtb_hbm.at[src_v.at[pj]],
                                         gbufs[pj], gsem[pj])

            def quad(t, carry):
                for gb in range(_NGBUF):
                    j = t * _NGBUF + gb
                    sb = gb % _NSBUF
                    gn = (gb + _NGBUF - 1) % _NGBUF

                    if (gb + _NGBUF - 1) % 2 == 0:
                        @pl.when(j + _NGBUF - 1 < _G)
                        def _():
                            pltpu.async_copy(
                                xt_sh.at[src_v.at[j + _NGBUF - 1]],
                                gbufs[gn], gsem[gn])
                    else:
                        @pl.when(jnp.logical_and(j + _NGBUF - 1 < _G,
                                                 c == 0))
                        def _():
                            pltpu.async_copy(
                                xta_hbm.at[src_v.at[j + _NGBUF - 1]],
                                gbufs[gn], gsem[gn])

                        @pl.when(jnp.logical_and(j + _NGBUF - 1 < _G,
                                                 c == 1))
                        def _():
                            pltpu.async_copy(
                                xtb_hbm.at[src_v.at[j + _NGBUF - 1]],
                                gbufs[gn], gsem[gn])

                    @pl.when(j >= _NSBUF)
                    def _():
                        pltpu.make_async_copy(
                            sbufs[sb], acc_sh.at[dst_v.at[j - _NSBUF]],
                            ssem[sb]).wait()

                    pltpu.make_async_copy(xt_sh.at[src_v.at[j]],
                                          gbufs[gb], gsem[gb]).wait()

                    def group16(gg, c2):
                        wv = ew_v[j, pl.ds(gg * 16, 16)]
                        for l in range(16):
                            e = gg * 16 + l
                            w = wv[l]
                            for g2 in range(2):
                                xb = gbufs[gb][e, pl.ds(g2 * 32, 32)]
                                xi = plsc.bitcast(xb, jnp.int32)
                                lo = plsc.bitcast(xi << 16, jnp.float32)
                                hi = plsc.bitcast(xi & mask_hi, jnp.float32)
                                sbufs[sb][e, pl.ds(g2 * 32, 16)] = lo * w
                                sbufs[sb][e, pl.ds(g2 * 32 + 16, 16)] = (
                                    hi * w)
                        return c2
                    lax.fori_loop(0, _CHUNK // 16, group16, 0)

                    pltpu.async_copy(sbufs[sb], acc_sh.at[dst_v.at[j]],
                                     ssem[sb], add=True)
                return carry
            lax.fori_loop(0, _G // _NGBUF, quad, 0)

            pltpu.make_async_copy(sbufs[0], acc_sh.at[dst_v.at[_G - 2]],
                                  ssem[0]).wait()
            pltpu.make_async_copy(sbufs[1], acc_sh.at[dst_v.at[_G - 1]],
                                  ssem[1]).wait()
            return carry0
        lax.fori_loop(0, _CPT // _G, group_body, 0)

        plsc.subcore_barrier()
        pltpu.sync_copy(acc_sh.at[pl.ds(base, _RPT)],
                        out_hbm.at[c, pl.ds(base, _RPT)])

    return k(xta, xtb, srcp, dstp, ewp)


def kernel(x, edge_index, edge_weight, W1, b1, W2, b2, W_out, b_out):
    pad = _EPAD - _E
    srcp = jnp.concatenate(
        [edge_index[0], jnp.zeros((pad,), jnp.int32)]).reshape(
            _NSUB, _CPT, _CHUNK)
    dstp = jnp.concatenate(
        [edge_index[1], jnp.zeros((pad,), jnp.int32)]).reshape(
            _NSUB, _CPT, _CHUNK)
    ewp = jnp.concatenate(
        [edge_weight, jnp.zeros((pad,), jnp.float32)]).reshape(
            _NSUB, _CPT, _CHUNK)

    w1t = W1.T
    w2t_p = W2.T[_PFULL, :]
    woutt_p = jnp.pad(W_out.T, ((0, 0), (0, _D - _OUT_DIM)))[_PFULL, :]
    boutp = jnp.pad(b_out, (0, _D - _OUT_DIM))

    xta, xtb = _stage_a(x, w1t, b1.reshape(1, _D))
    p = _spmm_sc(xta, xtb, srcp, dstp, ewp)
    xta, xtb = _stage_c(p[0, :_N], p[1, :_N], w2t_p, b2.reshape(1, _D))
    p = _spmm_sc(xta, xtb, srcp, dstp, ewp)
    out = _stage_d(p[0, :_N], p[1, :_N], woutt_p, boutp.reshape(1, _D))
    return out[:, :_OUT_DIM]


# speedup vs baseline: 1.4086x; 1.2222x over previous
import functools

import jax
import jax.numpy as jnp
import numpy as np
from jax import lax
from jax.experimental import pallas as pl
from jax.experimental.pallas import tpu as pltpu
from jax.experimental.pallas import tpu_sc as plsc

_N = 10000
_D = 128
_E = 320000
_OUT_DIM = 103
_MIN_NORM = 1e-15
_MAXNORM = 1.0 - 1e-5

_NCORE = 2
_NSUB = 16
_HD = _D // _NCORE
_CHUNK = 128
_CPT = 160
_G = 40
_EPW = _CHUNK * _CPT
_EPAD = _NSUB * _EPW
_NPAD = 10240
_RPT = _NPAD // _NSUB
_ZROWS = 128
_NGBUF = 4
_NSBUF = 2
_STG = 80

_QHALF = np.concatenate(
    [np.concatenate([g * 32 + 2 * np.arange(16),
                     g * 32 + 2 * np.arange(16) + 1]) for g in range(2)])
_PFULL = np.concatenate([_QHALF, 64 + _QHALF])

_BLK = 1000
_NBLK = _N // _BLK


def _norm(x):
    return jnp.maximum(jnp.sqrt(jnp.sum(x * x, axis=-1, keepdims=True)),
                       _MIN_NORM)


def _proj(x):
    n = _norm(x)
    return jnp.where(n > _MAXNORM, x / n * _MAXNORM, x)


def _artanh(x):
    x = jnp.clip(x, -1.0 + 1e-7, 1.0 - 1e-7)
    return 0.5 * jnp.log((1.0 + x) / (1.0 - x))


def _expmap0(u):
    n = _norm(u)
    return jnp.tanh(n) * u / n


def _logmap0(p):
    n = _norm(p)
    return _artanh(n) * p / n


def _mobius_add(x, y):
    x2 = jnp.sum(x * x, axis=-1, keepdims=True)
    y2 = jnp.sum(y * y, axis=-1, keepdims=True)
    xy = jnp.sum(x * y, axis=-1, keepdims=True)
    num = (1.0 + 2.0 * xy + y2) * x + (1.0 - x2) * y
    den = 1.0 + 2.0 * xy + x2 * y2
    return num / jnp.maximum(den, _MIN_NORM)


def _hyplinear_to_tangent(x, wt, b, proj_first):
    if proj_first:
        x = _proj(x)
    xn = _norm(x)
    mx = jnp.dot(x, wt, preferred_element_type=jnp.float32)
    mxn = _norm(mx)
    res = jnp.tanh(mxn / xn * _artanh(xn)) * mx / mxn
    mv = _proj(res)
    hyp_b = _proj(_expmap0(b))
    h = _proj(_mobius_add(mv, hyp_b))
    return _logmap0(h)


def _agg_to_hyp(p0, p1):
    agg = jnp.concatenate([p0, p1], axis=-1)
    h = _proj(_expmap0(agg))
    xt = jnp.maximum(_logmap0(h), 0.0)
    return _proj(_expmap0(xt))


def _stage_a_body(x_ref, wt_ref, b_ref, o0_ref, o1_ref):
    res = _hyplinear_to_tangent(x_ref[...], wt_ref[...], b_ref[...],
                                proj_first=True)
    o0_ref[...] = res[:, :_HD].astype(jnp.bfloat16)
    o1_ref[...] = res[:, _HD:].astype(jnp.bfloat16)


def _stage_c_body(p0_ref, p1_ref, wt_ref, b_ref, o0_ref, o1_ref):
    h = _agg_to_hyp(p0_ref[...], p1_ref[...])
    res = _hyplinear_to_tangent(h, wt_ref[...], b_ref[...],
                                proj_first=False)
    o0_ref[...] = res[:, :_HD].astype(jnp.bfloat16)
    o1_ref[...] = res[:, _HD:].astype(jnp.bfloat16)


def _stage_d_body(p0_ref, p1_ref, wt_ref, b_ref, o_ref):
    h = _agg_to_hyp(p0_ref[...], p1_ref[...])
    z = _logmap0(h)
    o_ref[...] = jnp.dot(z, wt_ref[...],
                         preferred_element_type=jnp.float32) + b_ref[...]


_row_spec = pl.BlockSpec((_BLK, _D), lambda i: (i, 0))
_half_spec = pl.BlockSpec((_BLK, _HD), lambda i: (i, 0))
_w_spec = pl.BlockSpec((_D, _D), lambda i: (0, 0))
_b_spec = pl.BlockSpec((1, _D), lambda i: (0, 0))
_out_struct = jax.ShapeDtypeStruct((_N, _D), jnp.float32)
_bf_struct = jax.ShapeDtypeStruct((_N, _HD), jnp.bfloat16)

_stage_a = pl.pallas_call(
    _stage_a_body, grid=(_NBLK,),
    in_specs=[_row_spec, _w_spec, _b_spec],
    out_specs=[_half_spec, _half_spec],
    out_shape=[_bf_struct, _bf_struct])

_stage_c = pl.pallas_call(
    _stage_c_body, grid=(_NBLK,),
    in_specs=[_half_spec, _half_spec, _w_spec, _b_spec],
    out_specs=[_half_spec, _half_spec],
    out_shape=[_bf_struct, _bf_struct])

_stage_d = pl.pallas_call(
    _stage_d_body, grid=(_NBLK,),
    in_specs=[_half_spec, _half_spec, _w_spec, _b_spec],
    out_specs=_row_spec, out_shape=_out_struct)


def _spmm_sc(xta, xtb, srcp, dstp, ewp):
    mesh = plsc.VectorSubcoreMesh(core_axis_name="c", subcore_axis_name="s")

    @functools.partial(
        pl.kernel,
        compiler_params=pltpu.CompilerParams(use_tc_tiling_on_sc=False,
                                             needs_layout_passes=False),
        out_type=jax.ShapeDtypeStruct((_NCORE, _NPAD, _HD), jnp.float32),
        mesh=mesh,
        scratch_types=[
            pltpu.VMEM((_G, _CHUNK), jnp.int32),
            pltpu.VMEM((_G, _CHUNK), jnp.int32),
            pltpu.VMEM((_G, _CHUNK), jnp.float32),
            [pltpu.VMEM((_CHUNK, _HD), jnp.bfloat16)] * _NGBUF,
            [pltpu.VMEM((_CHUNK, _HD), jnp.float32)] * _NSBUF,
            pltpu.VMEM_SHARED((_N, _HD), jnp.bfloat16),
            pltpu.VMEM_SHARED((_NPAD, _HD), jnp.float32),
            [pltpu.SemaphoreType.DMA] * _NGBUF,
            [pltpu.SemaphoreType.DMA] * _NSBUF,
        ],
    )
    def k(xta_hbm, xtb_hbm, srcp_hbm, dstp_hbm, ewp_hbm, out_hbm,
          src_v, dst_v, ew_v, gbufs, sbufs, xt_sh, acc_sh, gsem, ssem):
        c = lax.axis_index("c")
        s = lax.axis_index("s")
        base = s * _RPT

        for it in range(_RPT // _STG):
            row0 = base + it * _STG

            @pl.when(row0 < _N)
            def _():
                @pl.when(c == 0)
                def _():
                    pltpu.sync_copy(xta_hbm.at[pl.ds(row0, _STG)],
                                    xt_sh.at[pl.ds(row0, _STG)])

                @pl.when(c == 1)
                def _():
                    pltpu.sync_copy(xtb_hbm.at[pl.ds(row0, _STG)],
                                    xt_sh.at[pl.ds(row0, _STG)])

        def zfill(i, carry):
            for q in range(_HD // 16):
                sbufs[0][i, pl.ds(q * 16, 16)] = jnp.zeros((16,),
                                                           jnp.float32)
            return carry
        lax.fori_loop(0, _ZROWS, zfill, 0)
        for r in range(_RPT // _ZROWS):
            pltpu.sync_copy(sbufs[0], acc_sh.at[pl.ds(base + r * _ZROWS,
                                                      _ZROWS)])
        plsc.subcore_barrier()

        mask_hi = jnp.full((16,), -65536, jnp.int32)

        def group_body(g, carry0):
            goff = g * _G
            pltpu.sync_copy(srcp_hbm.at[s, pl.ds(goff, _G)], src_v)
            pltpu.sync_copy(dstp_hbm.at[s, pl.ds(goff, _G)], dst_v)
            pltpu.sync_copy(ewp_hbm.at[s, pl.ds(goff, _G)], ew_v)

            for pj in range(_NGBUF - 1):
                pltpu.async_copy(xt_sh.at[src_v.at[pj]], gbufs[pj],
                                 gsem[pj])

            def quad(t, carry):
                for gb in range(_NGBUF):
                    j = t * _NGBUF + gb
                    sb = gb % _NSBUF
                    gn = (gb + _NGBUF - 1) % _NGBUF

                    @pl.when(j + _NGBUF - 1 < _G)
                    def _():
                        pltpu.async_copy(
                            xt_sh.at[src_v.at[j + _NGBUF - 1]],
                            gbufs[gn], gsem[gn])

                    @pl.when(j >= _NSBUF)
                    def _():
                        pltpu.make_async_copy(
                            sbufs[sb], acc_sh.at[dst_v.at[j - _NSBUF]],
                            ssem[sb]).wait()

                    pltpu.make_async_copy(xt_sh.at[src_v.at[j]],
                                          gbufs[gb], gsem[gb]).wait()

                    @plsc.parallel_loop(0, _CHUNK // 16, unroll=2)
                    def group16(gg):
                        wv = ew_v[j, pl.ds(gg * 16, 16)]
                        for l in range(16):
                            e = gg * 16 + l
                            w = wv[l]
                            for g2 in range(2):
                                xb = gbufs[gb][e, pl.ds(g2 * 32, 32)]
                                xi = plsc.bitcast(xb, jnp.int32)
                                lo = plsc.bitcast(xi << 16, jnp.float32)
                                hi = plsc.bitcast(xi & mask_hi, jnp.float32)
                                sbufs[sb][e, pl.ds(g2 * 32, 16)] = lo * w
                                sbufs[sb][e, pl.ds(g2 * 32 + 16, 16)] = (
                                    hi * w)

                    pltpu.async_copy(sbufs[sb], acc_sh.at[dst_v.at[j]],
                                     ssem[sb], add=True)
                return carry
            lax.fori_loop(0, _G // _NGBUF, quad, 0)

            pltpu.make_async_copy(sbufs[0], acc_sh.at[dst_v.at[_G - 2]],
                                  ssem[0]).wait()
            pltpu.make_async_copy(sbufs[1], acc_sh.at[dst_v.at[_G - 1]],
                                  ssem[1]).wait()
            return carry0
        lax.fori_loop(0, _CPT // _G, group_body, 0)

        plsc.subcore_barrier()
        pltpu.sync_copy(acc_sh.at[pl.ds(base, _RPT)],
                        out_hbm.at[c, pl.ds(base, _RPT)])

    return k(xta, xtb, srcp, dstp, ewp)


def kernel(x, edge_index, edge_weight, W1, b1, W2, b2, W_out, b_out):
    pad = _EPAD - _E
    srcp = jnp.concatenate(
        [edge_index[0], jnp.zeros((pad,), jnp.int32)]).reshape(
            _NSUB, _CPT, _CHUNK)
    dstp = jnp.concatenate(
        [edge_index[1], jnp.zeros((pad,), jnp.int32)]).reshape(
            _NSUB, _CPT, _CHUNK)
    ewp = jnp.concatenate(
        [edge_weight, jnp.zeros((pad,), jnp.float32)]).reshape(
            _NSUB, _CPT, _CHUNK)

    w1t = W1.T
    w2t_p = W2.T[_PFULL, :]
    woutt_p = jnp.pad(W_out.T, ((0, 0), (0, _D - _OUT_DIM)))[_PFULL, :]
    boutp = jnp.pad(b_out, (0, _D - _OUT_DIM))

    xta, xtb = _stage_a(x, w1t, b1.reshape(1, _D))
    p = _spmm_sc(xta, xtb, srcp, dstp, ewp)
    xta, xtb = _stage_c(p[0, :_N], p[1, :_N], w2t_p, b2.reshape(1, _D))
    p = _spmm_sc(xta, xtb, srcp, dstp, ewp)
    out = _stage_d(p[0, :_N], p[1, :_N], woutt_p, boutp.reshape(1, _D))
    return out[:, :_OUT_DIM]


# speedup vs baseline: 1.6742x; 1.1886x over previous
import functools

import jax
import jax.numpy as jnp
import numpy as np
from jax import lax
from jax.experimental import pallas as pl
from jax.experimental.pallas import tpu as pltpu
from jax.experimental.pallas import tpu_sc as plsc

_N = 10000
_D = 128
_E = 320000
_OUT_DIM = 103
_MIN_NORM = 1e-15
_MAXNORM = 1.0 - 1e-5

_NCORE = 2
_NSUB = 16
_HD = _D // _NCORE
_CHUNK = 128
_CPT = 160
_G = 40
_EPW = _CHUNK * _CPT
_EPAD = _NSUB * _EPW
_NPAD = 10240
_RPT = _NPAD // _NSUB
_ZROWS = 128
_NGBUF = 4
_NSBUF = 2
_STG = 80

_QHALF = np.concatenate(
    [np.concatenate([g * 32 + 2 * np.arange(16),
                     g * 32 + 2 * np.arange(16) + 1]) for g in range(2)])
_PFULL = np.concatenate([_QHALF, 64 + _QHALF])

_BLK = 1000
_NBLK = _N // _BLK


def _norm(x):
    return jnp.maximum(jnp.sqrt(jnp.sum(x * x, axis=-1, keepdims=True)),
                       _MIN_NORM)


def _proj(x):
    n = _norm(x)
    return jnp.where(n > _MAXNORM, x / n * _MAXNORM, x)


def _artanh(x):
    x = jnp.clip(x, -1.0 + 1e-7, 1.0 - 1e-7)
    return 0.5 * jnp.log((1.0 + x) / (1.0 - x))


def _expmap0(u):
    n = _norm(u)
    return jnp.tanh(n) * u / n


def _logmap0(p):
    n = _norm(p)
    return _artanh(n) * p / n


def _mobius_add(x, y):
    x2 = jnp.sum(x * x, axis=-1, keepdims=True)
    y2 = jnp.sum(y * y, axis=-1, keepdims=True)
    xy = jnp.sum(x * y, axis=-1, keepdims=True)
    num = (1.0 + 2.0 * xy + y2) * x + (1.0 - x2) * y
    den = 1.0 + 2.0 * xy + x2 * y2
    return num / jnp.maximum(den, _MIN_NORM)


def _hyplinear_to_tangent(x, wt, b, proj_first):
    if proj_first:
        x = _proj(x)
    xn = _norm(x)
    mx = jnp.dot(x, wt, preferred_element_type=jnp.float32)
    mxn = _norm(mx)
    res = jnp.tanh(mxn / xn * _artanh(xn)) * mx / mxn
    mv = _proj(res)
    hyp_b = _proj(_expmap0(b))
    h = _proj(_mobius_add(mv, hyp_b))
    return _logmap0(h)


def _agg_to_hyp(p0, p1):
    agg = jnp.concatenate([p0, p1], axis=-1)
    h = _proj(_expmap0(agg))
    xt = jnp.maximum(_logmap0(h), 0.0)
    return _proj(_expmap0(xt))


def _stage_a_body(x_ref, wt_ref, b_ref, o0_ref, o1_ref):
    res = _hyplinear_to_tangent(x_ref[...], wt_ref[...], b_ref[...],
                                proj_first=True)
    o0_ref[...] = res[:, :_HD].astype(jnp.bfloat16)
    o1_ref[...] = res[:, _HD:].astype(jnp.bfloat16)


def _stage_c_body(p0_ref, p1_ref, wt_ref, b_ref, o0_ref, o1_ref):
    h = _agg_to_hyp(p0_ref[...], p1_ref[...])
    res = _hyplinear_to_tangent(h, wt_ref[...], b_ref[...],
                                proj_first=False)
    o0_ref[...] = res[:, :_HD].astype(jnp.bfloat16)
    o1_ref[...] = res[:, _HD:].astype(jnp.bfloat16)


def _stage_d_body(p0_ref, p1_ref, wt_ref, b_ref, o_ref):
    h = _agg_to_hyp(p0_ref[...], p1_ref[...])
    z = _logmap0(h)
    o_ref[...] = jnp.dot(z, wt_ref[...],
                         preferred_element_type=jnp.float32) + b_ref[...]


_row_spec = pl.BlockSpec((_BLK, _D), lambda i: (i, 0))
_half_spec = pl.BlockSpec((_BLK, _HD), lambda i: (i, 0))
_w_spec = pl.BlockSpec((_D, _D), lambda i: (0, 0))
_b_spec = pl.BlockSpec((1, _D), lambda i: (0, 0))
_out_struct = jax.ShapeDtypeStruct((_N, _D), jnp.float32)
_bf_struct = jax.ShapeDtypeStruct((_N, _HD), jnp.bfloat16)

_stage_a = pl.pallas_call(
    _stage_a_body, grid=(_NBLK,),
    in_specs=[_row_spec, _w_spec, _b_spec],
    out_specs=[_half_spec, _half_spec],
    out_shape=[_bf_struct, _bf_struct])

_stage_c = pl.pallas_call(
    _stage_c_body, grid=(_NBLK,),
    in_specs=[_half_spec, _half_spec, _w_spec, _b_spec],
    out_specs=[_half_spec, _half_spec],
    out_shape=[_bf_struct, _bf_struct])

_stage_d = pl.pallas_call(
    _stage_d_body, grid=(_NBLK,),
    in_specs=[_half_spec, _half_spec, _w_spec, _b_spec],
    out_specs=_row_spec, out_shape=_out_struct)


def _spmm_sc(xta, xtb, srcp, dstp, ewp):
    mesh = plsc.VectorSubcoreMesh(core_axis_name="c", subcore_axis_name="s")

    @functools.partial(
        pl.kernel,
        compiler_params=pltpu.CompilerParams(use_tc_tiling_on_sc=False,
                                             needs_layout_passes=False),
        out_type=jax.ShapeDtypeStruct((_NCORE, _NPAD, _HD), jnp.float32),
        mesh=mesh,
        scratch_types=[
            pltpu.VMEM((_G, _CHUNK), jnp.int32),
            pltpu.VMEM((_G, _CHUNK), jnp.int32),
            pltpu.VMEM((_G, _CHUNK), jnp.float32),
            [pltpu.VMEM((_CHUNK, _HD), jnp.bfloat16)] * _NGBUF,
            [pltpu.VMEM((_CHUNK, _HD), jnp.float32)] * _NSBUF,
            pltpu.VMEM_SHARED((_N, _HD), jnp.bfloat16),
            pltpu.VMEM_SHARED((_NPAD, _HD), jnp.float32),
            [pltpu.SemaphoreType.DMA] * _NGBUF,
            [pltpu.SemaphoreType.DMA] * _NSBUF,
        ],
    )
    def k(xta_hbm, xtb_hbm, srcp_hbm, dstp_hbm, ewp_hbm, out_hbm,
          src_v, dst_v, ew_v, gbufs, sbufs, xt_sh, acc_sh, gsem, ssem):
        c = lax.axis_index("c")
        s = lax.axis_index("s")
        base = s * _RPT

        for it in range(_RPT // _STG):
            row0 = base + it * _STG

            @pl.when(row0 < _N)
            def _():
                @pl.when(c == 0)
                def _():
                    pltpu.sync_copy(xta_hbm.at[pl.ds(row0, _STG)],
                                    xt_sh.at[pl.ds(row0, _STG)])

                @pl.when(c == 1)
                def _():
                    pltpu.sync_copy(xtb_hbm.at[pl.ds(row0, _STG)],
                                    xt_sh.at[pl.ds(row0, _STG)])

        def zfill(i, carry):
            for q in range(_HD // 16):
                sbufs[0][i, pl.ds(q * 16, 16)] = jnp.zeros((16,),
                                                           jnp.float32)
            return carry
        lax.fori_loop(0, _ZROWS, zfill, 0)
        for r in range(_RPT // _ZROWS):
            pltpu.sync_copy(sbufs[0], acc_sh.at[pl.ds(base + r * _ZROWS,
                                                      _ZROWS)])
        plsc.subcore_barrier()

        mask_hi = jnp.full((16,), -65536, jnp.int32)

        def group_body(g, carry0):
            goff = g * _G
            pltpu.sync_copy(srcp_hbm.at[s, pl.ds(goff, _G)], src_v)
            pltpu.sync_copy(dstp_hbm.at[s, pl.ds(goff, _G)], dst_v)
            pltpu.sync_copy(ewp_hbm.at[s, pl.ds(goff, _G)], ew_v)

            for pj in range(_NGBUF - 1):
                pltpu.async_copy(xt_sh.at[src_v.at[pj]], gbufs[pj],
                                 gsem[pj])

            def quad(t, carry):
                for gb in range(_NGBUF):
                    j = t * _NGBUF + gb
                    sb = gb % _NSBUF
                    gn = (gb + _NGBUF - 1) % _NGBUF

                    @pl.when(j + _NGBUF - 1 < _G)
                    def _():
                        pltpu.async_copy(
                            xt_sh.at[src_v.at[j + _NGBUF - 1]],
                            gbufs[gn], gsem[gn])

                    @pl.when(j >= _NSBUF)
                    def _():
                        pltpu.make_async_copy(
                            sbufs[sb], acc_sh.at[dst_v.at[j - _NSBUF]],
                            ssem[sb]).wait()

                    pltpu.make_async_copy(xt_sh.at[src_v.at[j]],
                                          gbufs[gb], gsem[gb]).wait()

                    @plsc.parallel_loop(0, _CHUNK // 16, unroll=4)
                    def group16(gg):
                        wv = ew_v[j, pl.ds(gg * 16, 16)]
                        for l in range(16):
                            e = gg * 16 + l
                            w = wv[l]
                            for g2 in range(2):
                                xb = gbufs[gb][e, pl.ds(g2 * 32, 32)]
                                xi = plsc.bitcast(xb, jnp.int32)
                                lo = plsc.bitcast(xi << 16, jnp.float32)
                                hi = plsc.bitcast(xi & mask_hi, jnp.float32)
                                sbufs[sb][e, pl.ds(g2 * 32, 16)] = lo * w
                                sbufs[sb][e, pl.ds(g2 * 32 + 16, 16)] = (
                                    hi * w)

                    pltpu.async_copy(sbufs[sb], acc_sh.at[dst_v.at[j]],
                                     ssem[sb], add=True)
                return carry
            lax.fori_loop(0, _G // _NGBUF, quad, 0)

            pltpu.make_async_copy(sbufs[0], acc_sh.at[dst_v.at[_G - 2]],
                                  ssem[0]).wait()
            pltpu.make_async_copy(sbufs[1], acc_sh.at[dst_v.at[_G - 1]],
                                  ssem[1]).wait()
            return carry0
        lax.fori_loop(0, _CPT // _G, group_body, 0)

        plsc.subcore_barrier()
        pltpu.sync_copy(acc_sh.at[pl.ds(base, _RPT)],
                        out_hbm.at[c, pl.ds(base, _RPT)])

    return k(xta, xtb, srcp, dstp, ewp)


def kernel(x, edge_index, edge_weight, W1, b1, W2, b2, W_out, b_out):
    pad = _EPAD - _E
    srcp = jnp.concatenate(
        [edge_index[0], jnp.zeros((pad,), jnp.int32)]).reshape(
            _NSUB, _CPT, _CHUNK)
    dstp = jnp.concatenate(
        [edge_index[1], jnp.zeros((pad,), jnp.int32)]).reshape(
            _NSUB, _CPT, _CHUNK)
    ewp = jnp.concatenate(
        [edge_weight, jnp.zeros((pad,), jnp.float32)]).reshape(
            _NSUB, _CPT, _CHUNK)

    w1t = W1.T
    w2t_p = W2.T[_PFULL, :]
    woutt_p = jnp.pad(W_out.T, ((0, 0), (0, _D - _OUT_DIM)))[_PFULL, :]
    boutp = jnp.pad(b_out, (0, _D - _OUT_DIM))

    xta, xtb = _stage_a(x, w1t, b1.reshape(1, _D))
    p = _spmm_sc(xta, xtb, srcp, dstp, ewp)
    xta, xtb = _stage_c(p[0, :_N], p[1, :_N], w2t_p, b2.reshape(1, _D))
    p = _spmm_sc(xta, xtb, srcp, dstp, ewp)
    out = _stage_d(p[0, :_N], p[1, :_N], woutt_p, boutp.reshape(1, _D))
    return out[:, :_OUT_DIM]
